# Initial kernel scaffold; baseline (speedup 1.0000x reference)
import jax
import jax.numpy as jnp
from jax.experimental import pallas as pl


def kernel(x, adj, PI, MUs, PREs, W1, W2):
    raise NotImplementedError("write your pallas kernel here")


# speedup vs baseline: 1.0782x; 1.0782x over previous
import functools

import jax
import jax.numpy as jnp
import numpy as np
from jax.experimental import pallas as pl
from jax.experimental.pallas import tpu as pltpu

_HIGHEST = jax.lax.Precision.HIGHEST


def _xw_kernel(x_ref, w_ref, y_ref):
    y_ref[...] = jnp.dot(x_ref[...], w_ref[...],
                         preferred_element_type=jnp.float32)


def _pass1_kernel(adj_ref, y_ref, pt_ref, mupt_ref, cvec_ref, w2_ref,
                  z_ref, probs_ref, *, nblocks, inv_n):
    i = pl.program_id(0)
    h = jnp.dot(adj_ref[...], y_ref[...],
                preferred_element_type=jnp.float32)
    ll = (jnp.dot(h, mupt_ref[...], precision=_HIGHEST,
                  preferred_element_type=jnp.float32)
          - 0.5 * jnp.dot(h * h, pt_ref[...], precision=_HIGHEST,
                          preferred_element_type=jnp.float32)
          + cvec_ref[...])
    rowmax = jnp.max(ll, axis=1, keepdims=True)
    s = jnp.sum(rowmax, axis=0, keepdims=True) * (-inv_n)

    @pl.when(i == 0)
    def _():
        probs_ref[...] = jnp.zeros_like(probs_ref)

    probs_ref[...] = probs_ref[...] + s

    x1 = jnp.maximum(h, 0.0)
    z_ref[...] = jnp.dot(x1, w2_ref[...], precision=_HIGHEST,
                         preferred_element_type=jnp.float32)


def _pass2_kernel(adj_ref, z_ref, out_ref):
    x2 = jnp.dot(adj_ref[...], z_ref[...],
                 preferred_element_type=jnp.float32)
    m = jnp.max(x2, axis=1, keepdims=True)
    sh = x2 - m
    lse = jnp.log(jnp.sum(jnp.exp(sh), axis=1, keepdims=True))
    out_ref[...] = sh - lse


def kernel(x, adj, PI, MUs, PREs, W1, W2):
    n, d_in = x.shape
    d_hid = W1.shape[1]
    d_out = W2.shape[1]
    k = PI.shape[0]

    cvec = (jnp.log(PI)
            + 0.5 * jnp.sum(jnp.log(PREs), axis=1)
            - 0.5 * jnp.sum(MUs * MUs * PREs, axis=1)
            - 8.0 * np.log(2.0 * np.pi)).reshape(1, k)
    pt = PREs.T
    mupt = (MUs * PREs).T

    bm_x = 2000
    y = pl.pallas_call(
        _xw_kernel,
        grid=(n // bm_x,),
        in_specs=[
            pl.BlockSpec((bm_x, d_in), lambda i: (i, 0)),
            pl.BlockSpec((d_in, d_hid), lambda i: (0, 0)),
        ],
        out_specs=pl.BlockSpec((bm_x, d_hid), lambda i: (i, 0)),
        out_shape=jax.ShapeDtypeStruct((n, d_hid), jnp.float32),
        compiler_params=pltpu.CompilerParams(
            dimension_semantics=("arbitrary",)),
    )(x, W1)

    bm = 400
    nblocks = n // bm
    z, probs2 = pl.pallas_call(
        functools.partial(_pass1_kernel, nblocks=nblocks, inv_n=1.0 / n),
        grid=(nblocks,),
        in_specs=[
            pl.BlockSpec((bm, n), lambda i: (i, 0)),
            pl.BlockSpec((n, d_hid), lambda i: (0, 0)),
            pl.BlockSpec((d_hid, k), lambda i: (0, 0)),
            pl.BlockSpec((d_hid, k), lambda i: (0, 0)),
            pl.BlockSpec((1, k), lambda i: (0, 0)),
            pl.BlockSpec((d_hid, d_out), lambda i: (0, 0)),
        ],
        out_specs=[
            pl.BlockSpec((bm, d_out), lambda i: (i, 0)),
            pl.BlockSpec((1, 1), lambda i: (0, 0)),
        ],
        out_shape=[
            jax.ShapeDtypeStruct((n, d_out), jnp.float32),
            jax.ShapeDtypeStruct((1, 1), jnp.float32),
        ],
        compiler_params=pltpu.CompilerParams(
            dimension_semantics=("arbitrary",)),
    )(adj, y, pt, mupt, cvec, W2)

    out = pl.pallas_call(
        _pass2_kernel,
        grid=(nblocks,),
        in_specs=[
            pl.BlockSpec((bm, n), lambda i: (i, 0)),
            pl.BlockSpec((n, d_out), lambda i: (0, 0)),
        ],
        out_specs=pl.BlockSpec((bm, d_out), lambda i: (i, 0)),
        out_shape=jax.ShapeDtypeStruct((n, d_out), jnp.float32),
        compiler_params=pltpu.CompilerParams(
            dimension_semantics=("arbitrary",)),
    )(adj, z)

    return (out, probs2[0, 0])


# speedup vs baseline: 1.1363x; 1.0539x over previous
import functools

import jax
import jax.numpy as jnp
import numpy as np
from jax.experimental import pallas as pl
from jax.experimental.pallas import tpu as pltpu

_HIGHEST = jax.lax.Precision.HIGHEST
_LOG2PI = float(np.log(2.0 * np.pi))


def _dot_t(a, b):
    return jax.lax.dot_general(a, b, (((1,), (1,)), ((), ())),
                               precision=_HIGHEST,
                               preferred_element_type=jnp.float32)


def _fused_kernel(x_ref, adj_ref, pi_ref, mus_ref, pres_ref, w1_ref, w2_ref,
                  out_ref, probs_ref, y_ref, z_ref,
                  *, a_steps, b_steps, bm_x, bm, inv_n):
    i = pl.program_id(0)

    @pl.when(i < a_steps)
    def _():
        y_ref[pl.ds(i * bm_x, bm_x), :] = jnp.dot(
            x_ref[...], w1_ref[...], preferred_element_type=jnp.float32)

    @pl.when((i >= a_steps) & (i < a_steps + b_steps))
    def _():
        h = jnp.dot(adj_ref[...], y_ref[...],
                    preferred_element_type=jnp.float32)
        pres = pres_ref[...]
        mus = mus_ref[...]
        ones = jnp.ones((1, mus.shape[1]), jnp.float32)
        cvec = (jnp.log(pi_ref[...])
                + 0.5 * _dot_t(ones, jnp.log(pres))
                - 0.5 * _dot_t(ones, mus * mus * pres)
                - 8.0 * _LOG2PI)
        ll = (_dot_t(h, mus * pres)
              - 0.5 * _dot_t(h * h, pres)
              + cvec)
        rowmax = jnp.max(ll, axis=1, keepdims=True)
        s = jnp.sum(rowmax, axis=0, keepdims=True) * (-inv_n)

        @pl.when(i == a_steps)
        def _():
            probs_ref[...] = jnp.zeros_like(probs_ref)

        probs_ref[...] = probs_ref[...] + s

        x1 = jnp.maximum(h, 0.0)
        z_ref[pl.ds((i - a_steps) * bm, bm), :] = jnp.dot(
            x1, w2_ref[...], precision=_HIGHEST,
            preferred_element_type=jnp.float32)

    @pl.when(i >= a_steps + b_steps)
    def _():
        x2 = jnp.dot(adj_ref[...], z_ref[...],
                     preferred_element_type=jnp.float32)
        m = jnp.max(x2, axis=1, keepdims=True)
        sh = x2 - m
        lse = jnp.log(jnp.sum(jnp.exp(sh), axis=1, keepdims=True))
        out_ref[...] = sh - lse


def kernel(x, adj, PI, MUs, PREs, W1, W2):
    n, d_in = x.shape
    d_hid = W1.shape[1]
    d_out = W2.shape[1]
    k = PI.shape[0]

    bm_x = 2000
    bm = 400
    a_steps = n // bm_x
    b_steps = n // bm
    ab = a_steps + b_steps

    out, probs2 = pl.pallas_call(
        functools.partial(_fused_kernel, a_steps=a_steps, b_steps=b_steps,
                          bm_x=bm_x, bm=bm, inv_n=1.0 / n),
        grid=(a_steps + 2 * b_steps,),
        in_specs=[
            pl.BlockSpec((bm_x, d_in),
                         lambda i: (jnp.minimum(i, a_steps - 1), 0)),
            pl.BlockSpec((bm, n),
                         lambda i: (jnp.where(
                             i < a_steps, 0,
                             jnp.where(i < ab, i - a_steps, i - ab)), 0)),
            pl.BlockSpec((1, k), lambda i: (0, 0)),
            pl.BlockSpec((k, d_hid), lambda i: (0, 0)),
            pl.BlockSpec((k, d_hid), lambda i: (0, 0)),
            pl.BlockSpec((d_in, d_hid), lambda i: (0, 0)),
            pl.BlockSpec((d_hid, d_out), lambda i: (0, 0)),
        ],
        out_specs=[
            pl.BlockSpec((bm, d_out),
                         lambda i: (jnp.maximum(i - ab, 0), 0)),
            pl.BlockSpec((1, 1), lambda i: (0, 0)),
        ],
        out_shape=[
            jax.ShapeDtypeStruct((n, d_out), jnp.float32),
            jax.ShapeDtypeStruct((1, 1), jnp.float32),
        ],
        scratch_shapes=[
            pltpu.VMEM((n, d_hid), jnp.float32),
            pltpu.VMEM((n, d_out), jnp.float32),
        ],
        compiler_params=pltpu.CompilerParams(
            dimension_semantics=("arbitrary",)),
    )(x, adj, PI.reshape(1, k), MUs, PREs, W1, W2)

    return (out, probs2[0, 0])
